# Initial kernel scaffold; baseline (speedup 1.0000x reference)
import functools

import jax
import jax.numpy as jnp
from jax.experimental import pallas as pl
from jax.experimental.pallas import tpu as pltpu
from jax.experimental.pallas import tpu_sc as plsc

BATCH = 4096
SEQ = 200
EMBED = 64
NFLAT = BATCH * SEQ
WIN = 128
LANES = 16


def _prep_body(seg_label_ref, seg_table_ref, pos_table_ref, comb_ref, cidx_ref):
    pos = pos_table_ref[...]
    seg = seg_table_ref[...]
    comb_ref[...] = pos + seg[:, None, :]
    s_iota = jax.lax.broadcasted_iota(jnp.int32, cidx_ref.shape, 1)
    cidx_ref[...] = seg_label_ref[...] * SEQ + s_iota


_prep = pl.pallas_call(
    _prep_body,
    out_shape=[
        jax.ShapeDtypeStruct((3, SEQ, EMBED), jnp.float32),
        jax.ShapeDtypeStruct((BATCH, SEQ), jnp.int32),
    ],
)


def _embed_kernel(tok_hbm, comb_hbm, seq_hbm, cidx_hbm, out_hbm,
                  comb_scr, sem_a, sem_b):
    def body(seq_vmem, cidx_vmem, out_vmem):
        cp_a = pltpu.async_copy(tok_hbm.at[seq_vmem.at[0]], out_vmem, sem_a)
        cp_b = pltpu.async_copy(comb_hbm.at[cidx_vmem.at[0]], comb_scr, sem_b)
        cp_a.wait()
        cp_b.wait()

        @pl.loop(0, WIN)
        def _(r):
            for j in range(EMBED // LANES):
                slc = (pl.ds(r, 1), pl.ds(j * LANES, LANES))
                out_vmem.at[slc][...] = (
                    out_vmem.at[slc][...] + comb_scr.at[slc][...]
                )

    pltpu.emit_pipeline(
        body,
        grid=(NFLAT // WIN,),
        in_specs=[
            pl.BlockSpec((1, WIN), lambda i: (0, i)),
            pl.BlockSpec((1, WIN), lambda i: (0, i)),
        ],
        out_specs=[pl.BlockSpec((WIN, EMBED), lambda i: (i, 0))],
        core_axis_name=("core", "subcore"),
        dimension_semantics=(pltpu.PARALLEL,),
    )(seq_hbm, cidx_hbm, out_hbm)


def _make_embed():
    mesh = plsc.VectorSubcoreMesh(
        core_axis_name="core", subcore_axis_name="subcore"
    )
    return pl.kernel(
        _embed_kernel,
        out_type=jax.ShapeDtypeStruct((NFLAT, EMBED), jnp.float32),
        mesh=mesh,
        scratch_types=[
            pltpu.VMEM((WIN, EMBED), jnp.float32),
            pltpu.SemaphoreType.DMA,
            pltpu.SemaphoreType.DMA,
        ],
    )


_embed = _make_embed()


@jax.jit
def kernel(sequence, segment_label, token_table, segment_table, pos_table):
    comb, cidx = _prep(
        segment_label.astype(jnp.int32), segment_table, pos_table
    )
    out = _embed(
        token_table,
        comb.reshape(3 * SEQ, EMBED),
        sequence.astype(jnp.int32).reshape(1, NFLAT),
        cidx.reshape(1, NFLAT),
    )
    return out.reshape(BATCH, SEQ, EMBED)


# speedup vs baseline: 1.8682x; 1.8682x over previous
import functools

import jax
import jax.numpy as jnp
from jax.experimental import pallas as pl
from jax.experimental.pallas import tpu as pltpu
from jax.experimental.pallas import tpu_sc as plsc

BATCH = 4096
SEQ = 200
EMBED = 64
NFLAT = BATCH * SEQ
WIN = 128
LANES = 16


def _prep_body(seg_label_ref, seg_table_ref, pos_table_ref, comb_ref, cidx_ref):
    pos = pos_table_ref[...]
    seg = seg_table_ref[...]
    comb_ref[...] = pos + seg[:, None, :]
    s_iota = jax.lax.broadcasted_iota(jnp.int32, cidx_ref.shape, 1)
    cidx_ref[...] = seg_label_ref[...] * SEQ + s_iota


_prep = pl.pallas_call(
    _prep_body,
    out_shape=[
        jax.ShapeDtypeStruct((3, SEQ, EMBED), jnp.float32),
        jax.ShapeDtypeStruct((BATCH, SEQ), jnp.int32),
    ],
)


def _embed_kernel(tok_hbm, comb_hbm, seq_hbm, cidx_hbm, out_hbm,
                  comb_scr, sem_a, sem_b):
    def body(seq_vmem, cidx_vmem, out_vmem):
        cp_a = pltpu.async_copy(tok_hbm.at[seq_vmem.at[0]], out_vmem, sem_a)
        cp_b = pltpu.async_copy(comb_hbm.at[cidx_vmem.at[0]], comb_scr, sem_b)
        cp_a.wait()
        cp_b.wait()

        @pl.loop(0, WIN)
        def _(r):
            for j in range(EMBED // LANES):
                slc = (pl.ds(r, 1), pl.ds(j * LANES, LANES))
                out_vmem.at[slc][...] = (
                    out_vmem.at[slc][...] + comb_scr.at[slc][...]
                )

    pltpu.emit_pipeline(
        body,
        grid=(NFLAT // WIN,),
        in_specs=[
            pl.BlockSpec((1, WIN), lambda i: (0, i)),
            pl.BlockSpec((1, WIN), lambda i: (0, i)),
        ],
        out_specs=[pl.BlockSpec((WIN, EMBED), lambda i: (i, 0))],
        core_axis_name=("core", "subcore"),
        dimension_semantics=(pltpu.PARALLEL,),
    )(seq_hbm, cidx_hbm, out_hbm)


def _make_embed():
    mesh = plsc.VectorSubcoreMesh(
        core_axis_name="core", subcore_axis_name="subcore"
    )
    return pl.kernel(
        _embed_kernel,
        out_type=jax.ShapeDtypeStruct((NFLAT, EMBED), jnp.float32),
        mesh=mesh,
        compiler_params=pltpu.CompilerParams(use_tc_tiling_on_sc=False),
        scratch_types=[
            pltpu.VMEM((WIN, EMBED), jnp.float32),
            pltpu.SemaphoreType.DMA,
            pltpu.SemaphoreType.DMA,
        ],
    )


_embed = _make_embed()


@jax.jit
def kernel(sequence, segment_label, token_table, segment_table, pos_table):
    comb, cidx = _prep(
        segment_label.astype(jnp.int32), segment_table, pos_table
    )
    out = _embed(
        token_table,
        comb.reshape(3 * SEQ, EMBED),
        sequence.astype(jnp.int32).reshape(1, NFLAT),
        cidx.reshape(1, NFLAT),
    )
    return out.reshape(BATCH, SEQ, EMBED)


# speedup vs baseline: 1.9089x; 1.0218x over previous
import functools

import jax
import jax.numpy as jnp
from jax.experimental import pallas as pl
from jax.experimental.pallas import tpu as pltpu
from jax.experimental.pallas import tpu_sc as plsc

BATCH = 4096
SEQ = 200
EMBED = 64
NFLAT = BATCH * SEQ
WIN = 128
LANES = 16


def _prep_body(seg_label_ref, seg_table_ref, pos_table_ref, comb_ref, cidx_ref):
    pos = pos_table_ref[...]
    seg = seg_table_ref[...]
    comb_ref[...] = pos + seg[:, None, :]
    s_iota = jax.lax.broadcasted_iota(jnp.int32, cidx_ref.shape, 1)
    cidx_ref[...] = seg_label_ref[...] * SEQ + s_iota


_prep = pl.pallas_call(
    _prep_body,
    out_shape=[
        jax.ShapeDtypeStruct((3, SEQ, EMBED), jnp.float32),
        jax.ShapeDtypeStruct((BATCH, SEQ), jnp.int32),
    ],
)


def _embed_kernel(tok_hbm, comb_hbm, seq_hbm, cidx_hbm, out_hbm,
                  comb_tbl, comb_scr, sem_a, sem_b):
    @pl.when(jax.lax.axis_index("subcore") == 0)
    def _():
        pltpu.sync_copy(comb_hbm, comb_tbl)

    plsc.subcore_barrier()

    def body(seq_vmem, cidx_vmem, out_vmem):
        cp_a = pltpu.async_copy(tok_hbm.at[seq_vmem.at[0]], out_vmem, sem_a)
        cp_b = pltpu.async_copy(comb_tbl.at[cidx_vmem.at[0]], comb_scr, sem_b)
        cp_a.wait()
        cp_b.wait()

        @pl.loop(0, WIN)
        def _(r):
            for j in range(EMBED // LANES):
                slc = (pl.ds(r, 1), pl.ds(j * LANES, LANES))
                out_vmem.at[slc][...] = (
                    out_vmem.at[slc][...] + comb_scr.at[slc][...]
                )

    pltpu.emit_pipeline(
        body,
        grid=(NFLAT // WIN,),
        in_specs=[
            pl.BlockSpec((1, WIN), lambda i: (0, i)),
            pl.BlockSpec((1, WIN), lambda i: (0, i)),
        ],
        out_specs=[pl.BlockSpec((WIN, EMBED), lambda i: (i, 0))],
        core_axis_name=("core", "subcore"),
        dimension_semantics=(pltpu.PARALLEL,),
    )(seq_hbm, cidx_hbm, out_hbm)


def _make_embed():
    mesh = plsc.VectorSubcoreMesh(
        core_axis_name="core", subcore_axis_name="subcore"
    )
    return pl.kernel(
        _embed_kernel,
        out_type=jax.ShapeDtypeStruct((NFLAT, EMBED), jnp.float32),
        mesh=mesh,
        compiler_params=pltpu.CompilerParams(use_tc_tiling_on_sc=False),
        scratch_types=[
            pltpu.VMEM_SHARED((3 * SEQ, EMBED), jnp.float32),
            pltpu.VMEM((WIN, EMBED), jnp.float32),
            pltpu.SemaphoreType.DMA,
            pltpu.SemaphoreType.DMA,
        ],
    )


_embed = _make_embed()


@jax.jit
def kernel(sequence, segment_label, token_table, segment_table, pos_table):
    comb, cidx = _prep(
        segment_label.astype(jnp.int32), segment_table, pos_table
    )
    out = _embed(
        token_table,
        comb.reshape(3 * SEQ, EMBED),
        sequence.astype(jnp.int32).reshape(1, NFLAT),
        cidx.reshape(1, NFLAT),
    )
    return out.reshape(BATCH, SEQ, EMBED)


# speedup vs baseline: 2.7385x; 1.4346x over previous
import functools

import jax
import jax.numpy as jnp
from jax.experimental import pallas as pl
from jax.experimental.pallas import tpu as pltpu
from jax.experimental.pallas import tpu_sc as plsc

BATCH = 4096
SEQ = 200
EMBED = 64
NFLAT = BATCH * SEQ
WIN = 128
LANES = 16


def _prep_body(seg_label_ref, seg_table_ref, pos_table_ref, comb_ref, cidx_ref):
    pos = pos_table_ref[...]
    seg = seg_table_ref[...]
    comb_ref[...] = pos + seg[:, None, :]
    s_iota = jax.lax.broadcasted_iota(jnp.int32, cidx_ref.shape, 1)
    cidx_ref[...] = seg_label_ref[...] * SEQ + s_iota


_prep = pl.pallas_call(
    _prep_body,
    out_shape=[
        jax.ShapeDtypeStruct((3, SEQ, EMBED), jnp.float32),
        jax.ShapeDtypeStruct((BATCH, SEQ), jnp.int32),
    ],
)


NWORK = 32
STEPS = NFLAT // WIN // NWORK


def _embed_kernel(tok_hbm, comb_hbm, seq_hbm, cidx_hbm, out_hbm,
                  comb_tbl,
                  idx_a0, idx_a1, idx_b0, idx_b1,
                  tok0, tok1, cmb0, cmb1, ob0, ob1,
                  sem_i0, sem_i1, sem_gt0, sem_gt1,
                  sem_gc0, sem_gc1, sem_o0, sem_o1):
    @pl.when(jax.lax.axis_index("subcore") == 0)
    def _():
        pltpu.sync_copy(comb_hbm, comb_tbl)

    plsc.subcore_barrier()

    wid = jax.lax.axis_index("subcore") * 2 + jax.lax.axis_index("core")
    base = wid * STEPS

    slots = (
        (idx_a0, idx_b0, tok0, cmb0, ob0, sem_i0, sem_gt0, sem_gc0, sem_o0),
        (idx_a1, idx_b1, tok1, cmb1, ob1, sem_i1, sem_gt1, sem_gc1, sem_o1),
    )

    def issue_gathers(slot):
        idx_a, idx_b, tok, cmb, _, _, sem_gt, sem_gc, _ = slot
        pltpu.async_copy(tok_hbm.at[idx_a], tok, sem_gt)
        pltpu.async_copy(comb_tbl.at[idx_b], cmb, sem_gc)

    def wait_gathers(slot):
        idx_a, idx_b, tok, cmb, _, _, sem_gt, sem_gc, _ = slot
        pltpu.make_async_copy(tok_hbm.at[idx_a], tok, sem_gt).wait()
        pltpu.make_async_copy(comb_tbl.at[idx_b], cmb, sem_gc).wait()

    def issue_idx(k, slot):
        idx_a, idx_b, _, _, _, sem_i, _, _, _ = slot
        off = (base + k) * WIN
        pltpu.async_copy(seq_hbm.at[pl.ds(off, WIN)], idx_a, sem_i)
        pltpu.async_copy(cidx_hbm.at[pl.ds(off, WIN)], idx_b, sem_i)

    def wait_idx(slot):
        idx_a, idx_b, _, _, _, sem_i, _, _, _ = slot
        pltpu.make_async_copy(seq_hbm.at[pl.ds(0, WIN)], idx_a, sem_i).wait()
        pltpu.make_async_copy(cidx_hbm.at[pl.ds(0, WIN)], idx_b, sem_i).wait()

    def wait_out(k, slot):
        _, _, _, _, ob, _, _, _, sem_o = slot
        row0 = (base + k) * WIN
        pltpu.make_async_copy(
            ob, out_hbm.at[pl.ds(row0, WIN)], sem_o
        ).wait()

    for s in range(2):
        idx_a, idx_b, _, _, _, _, _, _, _ = slots[s]
        off = (base + s) * WIN
        pltpu.sync_copy(seq_hbm.at[pl.ds(off, WIN)], idx_a)
        pltpu.sync_copy(cidx_hbm.at[pl.ds(off, WIN)], idx_b)
        issue_gathers(slots[s])

    def stage(k, s):
        slot = slots[s]
        _, _, tok, cmb, ob, _, _, _, sem_o = slot
        wait_gathers(slot)

        @pl.when(k + 2 < STEPS)
        def _():
            issue_idx(k + 2, slot)

        @pl.when(k >= 2)
        def _():
            wait_out(k - 2, slot)

        @pl.loop(0, WIN, step=4)
        def _(r):
            for rr in range(4):
                for j in range(EMBED // LANES):
                    slc = (pl.ds(r + rr, 1), pl.ds(j * LANES, LANES))
                    ob.at[slc][...] = tok.at[slc][...] + cmb.at[slc][...]

        row0 = (base + k) * WIN
        pltpu.async_copy(ob, out_hbm.at[pl.ds(row0, WIN)], sem_o)

        @pl.when(k + 2 < STEPS)
        def _():
            wait_idx(slot)
            issue_gathers(slot)

    @pl.loop(0, STEPS, step=2)
    def _(k):
        stage(k, 0)
        stage(k + 1, 1)

    wait_out(STEPS - 2, slots[0])
    wait_out(STEPS - 1, slots[1])


def _make_embed():
    mesh = plsc.VectorSubcoreMesh(
        core_axis_name="core", subcore_axis_name="subcore"
    )
    return pl.kernel(
        _embed_kernel,
        out_type=jax.ShapeDtypeStruct((NFLAT, EMBED), jnp.float32),
        mesh=mesh,
        compiler_params=pltpu.CompilerParams(use_tc_tiling_on_sc=False),
        scratch_types=[
            pltpu.VMEM_SHARED((3 * SEQ, EMBED), jnp.float32),
            pltpu.VMEM((WIN,), jnp.int32),
            pltpu.VMEM((WIN,), jnp.int32),
            pltpu.VMEM((WIN,), jnp.int32),
            pltpu.VMEM((WIN,), jnp.int32),
            pltpu.VMEM((WIN, EMBED), jnp.float32),
            pltpu.VMEM((WIN, EMBED), jnp.float32),
            pltpu.VMEM((WIN, EMBED), jnp.float32),
            pltpu.VMEM((WIN, EMBED), jnp.float32),
            pltpu.VMEM((WIN, EMBED), jnp.float32),
            pltpu.VMEM((WIN, EMBED), jnp.float32),
        ] + [pltpu.SemaphoreType.DMA] * 8,
    )


_embed = _make_embed()


@jax.jit
def kernel(sequence, segment_label, token_table, segment_table, pos_table):
    comb, cidx = _prep(
        segment_label.astype(jnp.int32), segment_table, pos_table
    )
    out = _embed(
        token_table,
        comb.reshape(3 * SEQ, EMBED),
        sequence.astype(jnp.int32).reshape(NFLAT),
        cidx.reshape(NFLAT),
    )
    return out.reshape(BATCH, SEQ, EMBED)
